# Initial kernel scaffold; baseline (speedup 1.0000x reference)
import functools

import jax
import jax.numpy as jnp
from jax import lax
from jax.experimental import pallas as pl
from jax.experimental.pallas import tpu as pltpu
from jax.experimental.pallas import tpu_sc as plsc

NC = 2
NS = 16
NW = NC * NS

E = 320000
D = 128
EW = E // NW
C = 80
NCHUNK = EW // C


def _sc_kernel(z_hbm, src_hbm, dst_hbm, out_hbm,
               sidx_v, didx_v, srows_v, drows_v, out_v, sem_s, sem_d):
    wid = lax.axis_index("s") * NC + lax.axis_index("c")
    base = wid * EW

    def chunk_body(k, _):
        off = base + k * C
        pltpu.sync_copy(src_hbm.at[pl.ds(off, C)], sidx_v)
        pltpu.sync_copy(dst_hbm.at[pl.ds(off, C)], didx_v)
        cs = pltpu.async_copy(z_hbm.at[sidx_v], srows_v, sem_s)
        cd = pltpu.async_copy(z_hbm.at[didx_v], drows_v, sem_d)
        cs.wait()
        cd.wait()

        lanes = lax.iota(jnp.int32, 16)

        def group_body(g, _):
            rows = g * 16 + lanes

            def dim_body(d, acc):
                for u in range(4):
                    col = jnp.full((16,), d * 4 + u, dtype=jnp.int32)
                    sv = plsc.load_gather(srows_v, [rows, col])
                    dv = plsc.load_gather(drows_v, [rows, col])
                    acc = acc + sv * dv
                return acc

            acc = lax.fori_loop(0, D // 4, dim_body,
                                jnp.zeros((16,), jnp.float32))
            out_v[pl.ds(g * 16, 16)] = acc
            return 0

        lax.fori_loop(0, C // 16, group_body, 0)
        pltpu.sync_copy(out_v, out_hbm.at[pl.ds(off, C)])
        return 0

    lax.fori_loop(0, NCHUNK, chunk_body, 0)


@jax.jit
def _run(z, src, dst):
    mesh = plsc.VectorSubcoreMesh(core_axis_name="c", subcore_axis_name="s")
    return pl.kernel(
        _sc_kernel,
        out_type=jax.ShapeDtypeStruct((E,), jnp.float32),
        mesh=mesh,
        scratch_types=[
            pltpu.VMEM((C,), jnp.int32),
            pltpu.VMEM((C,), jnp.int32),
            pltpu.VMEM((C, D), jnp.float32),
            pltpu.VMEM((C, D), jnp.float32),
            pltpu.VMEM((C,), jnp.float32),
            pltpu.SemaphoreType.DMA,
            pltpu.SemaphoreType.DMA,
        ],
    )(z, src, dst)


def kernel(z, edge_index):
    src = edge_index[0, :].astype(jnp.int32)
    dst = edge_index[1, :].astype(jnp.int32)
    return _run(z, src, dst)


# speedup vs baseline: 7.7036x; 7.7036x over previous
import functools

import jax
import jax.numpy as jnp
from jax import lax
from jax.experimental import pallas as pl
from jax.experimental.pallas import tpu as pltpu
from jax.experimental.pallas import tpu_sc as plsc

NC = 2
NS = 16
NW = NC * NS

E = 320000
D = 128
EW = E // NW
C = 80
NCHUNK = EW // C


def _sc_kernel(z_hbm, src_hbm, dst_hbm, out_hbm,
               sidx0, sidx1, didx0, didx1,
               srows0, srows1, drows0, drows1,
               out0, out1, tr_v,
               sem_i0, sem_i1, sem_g0, sem_g1, sem_o0, sem_o1):
    SI = (sidx0, sidx1)
    DI = (didx0, didx1)
    SR = (srows0, srows1)
    DR = (drows0, drows1)
    OV = (out0, out1)
    SEMI = (sem_i0, sem_i1)
    SEMG = (sem_g0, sem_g1)
    SEMO = (sem_o0, sem_o1)

    wid = lax.axis_index("s") * NC + lax.axis_index("c")
    base = wid * EW
    col0 = lax.iota(jnp.int32, 16) * 16

    def issue_idx(b, k):
        off = base + k * C
        pltpu.async_copy(src_hbm.at[pl.ds(off, C)], SI[b], SEMI[b])
        pltpu.async_copy(dst_hbm.at[pl.ds(off, C)], DI[b], SEMI[b])

    def wait_idx(b, k):
        off = base + k * C
        pltpu.make_async_copy(src_hbm.at[pl.ds(off, C)], SI[b], SEMI[b]).wait()
        pltpu.make_async_copy(dst_hbm.at[pl.ds(off, C)], DI[b], SEMI[b]).wait()

    def gather(b):
        pltpu.async_copy(z_hbm.at[SI[b]], SR[b], SEMG[b])
        pltpu.async_copy(z_hbm.at[DI[b]], DR[b], SEMG[b])

    def wait_gather(b):
        pltpu.make_async_copy(z_hbm.at[SI[b]], SR[b], SEMG[b]).wait()
        pltpu.make_async_copy(z_hbm.at[DI[b]], DR[b], SEMG[b]).wait()

    def wait_out(b, k):
        off = base + k * C
        pltpu.make_async_copy(OV[b], out_hbm.at[pl.ds(off, C)], SEMO[b]).wait()

    def compute(b, k):
        off = base + k * C
        srows_v = SR[b]
        drows_v = DR[b]
        out_v = OV[b]

        def group_body(g, _):
            e0 = g * 16
            for e in range(16):
                acc = (srows_v[e0 + e, pl.ds(0, 16)]
                       * drows_v[e0 + e, pl.ds(0, 16)])
                for j in range(1, D // 16):
                    acc = acc + (srows_v[e0 + e, pl.ds(j * 16, 16)]
                                 * drows_v[e0 + e, pl.ds(j * 16, 16)])
                tr_v[pl.ds(e * 16, 16)] = acc
            res = plsc.load_gather(tr_v, [col0])
            for j in range(1, 16):
                res = res + plsc.load_gather(tr_v, [col0 + j])
            out_v[pl.ds(e0, 16)] = res
            return 0

        lax.fori_loop(0, C // 16, group_body, 0)
        pltpu.async_copy(out_v, out_hbm.at[pl.ds(off, C)], SEMO[b])

    def step(k, b):
        nb = 1 - b
        wait_idx(nb, k + 1)
        gather(nb)
        wait_gather(b)

        @pl.when(k + 2 < NCHUNK)
        def _():
            issue_idx(b, k + 2)

        @pl.when(k >= 2)
        def _():
            wait_out(b, k - 2)

        compute(b, k)

    issue_idx(0, 0)
    wait_idx(0, 0)
    gather(0)
    issue_idx(1, 1)

    def pair_body(i, _):
        step(2 * i, 0)
        step(2 * i + 1, 1)
        return 0

    lax.fori_loop(0, (NCHUNK - 1) // 2, pair_body, 0)

    kt = NCHUNK - 1
    wait_gather(0)
    wait_out(0, kt - 2)
    compute(0, kt)
    wait_out(1, kt - 1)
    wait_out(0, kt)


@jax.jit
def _run(z, src, dst):
    mesh = plsc.VectorSubcoreMesh(core_axis_name="c", subcore_axis_name="s")
    return pl.kernel(
        _sc_kernel,
        out_type=jax.ShapeDtypeStruct((E,), jnp.float32),
        mesh=mesh,
        compiler_params=pltpu.CompilerParams(needs_layout_passes=False),
        scratch_types=[
            pltpu.VMEM((C,), jnp.int32),
            pltpu.VMEM((C,), jnp.int32),
            pltpu.VMEM((C,), jnp.int32),
            pltpu.VMEM((C,), jnp.int32),
            pltpu.VMEM((C, D), jnp.float32),
            pltpu.VMEM((C, D), jnp.float32),
            pltpu.VMEM((C, D), jnp.float32),
            pltpu.VMEM((C, D), jnp.float32),
            pltpu.VMEM((C,), jnp.float32),
            pltpu.VMEM((C,), jnp.float32),
            pltpu.VMEM((256,), jnp.float32),
            pltpu.SemaphoreType.DMA,
            pltpu.SemaphoreType.DMA,
            pltpu.SemaphoreType.DMA,
            pltpu.SemaphoreType.DMA,
            pltpu.SemaphoreType.DMA,
            pltpu.SemaphoreType.DMA,
        ],
    )(z, src, dst)


def kernel(z, edge_index):
    src = edge_index[0, :].astype(jnp.int32)
    dst = edge_index[1, :].astype(jnp.int32)
    return _run(z, src, dst)
